# baseline (speedup 1.0000x reference)
import functools
import math

import jax
import jax.numpy as jnp
from jax.experimental import pallas as pl
from jax.experimental.pallas import tpu as pltpu

N = 1024
H = 128
K = 16
L = 3
ND = 3
BIG = 1e30
_SCALE = math.sqrt(128.0)
_PREC = jax.lax.Precision.DEFAULT


def _dot(a, b, precision=_PREC):
    return jnp.dot(a, b, preferred_element_type=jnp.float32, precision=precision)


def _dot_nt(a, b, precision=_PREC):
    return jax.lax.dot_general(
        a, b, (((1,), (1,)), ((), ())),
        preferred_element_type=jnp.float32, precision=precision)


def _gelu(x):
    return jax.nn.gelu(x)


def _layer_norm(x, sc, bi):
    m = jnp.mean(x, axis=-1, keepdims=True)
    v = jnp.mean((x - m) ** 2, axis=-1, keepdims=True)
    return (x - m) / jnp.sqrt(v + 1e-6) * sc + bi


def _cloud_kernel(pts_ref, posT_ref, Win_ref, bin_ref,
                  Wqkv_ref, bqkv_ref,
                  Wp_ref, bp_ref, WpdT_ref, wa_ref,
                  Wo_ref, bo_ref, lns_ref, lnb_ref,
                  out_ref, A_s, madd_s):
    pts = pts_ref[0]
    pos = pts[:, :ND]
    x = _dot(pts, Win_ref[...]) + bin_ref[...]

    px = pos[:, 0:1]
    py = pos[:, 1:2]
    pz = pos[:, 2:3]
    rx = posT_ref[0, 0:1, :]
    ry = posT_ref[0, 1:2, :]
    rz = posT_ref[0, 2:3, :]
    dx = px - rx
    dy = py - ry
    dz = pz - rz
    col = jax.lax.broadcasted_iota(jnp.int32, (N, N), 1)
    row = jax.lax.broadcasted_iota(jnp.int32, (N, N), 0)
    A_s[...] = jnp.where(col == row, BIG, dx * dx + dy * dy + dz * dz)

    def _select_one(_, carry):
        d = A_s[...]
        m = jnp.min(d, axis=1, keepdims=True)
        idx = jnp.min(jnp.where(d == m, col, N), axis=1, keepdims=True)
        A_s[...] = jnp.where(col == idx, BIG, d)
        return carry

    for _ in range(K - 1):
        _select_one(0, 0)
    madd_s[...] = jnp.where(A_s[...] == BIG, 0.0, -BIG)
    madd = madd_s[...]

    for i in range(L):
        qkv = _dot(x, Wqkv_ref[i]) + bqkv_ref[i]
        q = qkv[:, :H]
        k = qkv[:, H:2 * H]
        v = qkv[:, 2 * H:]
        pe = _dot(pos, Wp_ref[i]) + bp_ref[i]
        u = (q + pe) * wa_ref[i]
        kk = k + pe
        w = _dot(u, WpdT_ref[i])
        Araw = _dot_nt(u, kk)
        m = jnp.max(
            Araw + (w[:, 0:1] * rx + w[:, 1:2] * ry + w[:, 2:3] * rz) + madd,
            axis=1, keepdims=True)
        p_un = jnp.exp(
            Araw + (w[:, 0:1] * rx + w[:, 1:2] * ry + w[:, 2:3] * rz) + madd
            - m)
        ssum = jnp.sum(p_un, axis=1, keepdims=True)
        out = _dot(p_un, v) / ssum
        out = _dot(out, Wo_ref[i]) + bo_ref[i]
        x = x + _gelu(out)
        x = _layer_norm(x, lns_ref[i], lnb_ref[i])

    out_ref[0] = jnp.max(x, axis=0, keepdims=True)


def _encoder_kernel(x_ref, Wq_ref, Wk_ref, Wv_ref, W1_ref, W2_ref,
                    bq_ref, bk_ref, bv_ref, b1_ref, b2_ref,
                    lnfs_ref, lnfb_ref, out_ref):
    nb = x_ref.shape[0]
    for b in range(nb):
        xb = x_ref[b]
        q = _dot(xb, Wq_ref[...]) + bq_ref[...]
        k = _dot(xb, Wk_ref[...]) + bk_ref[...]
        v = _dot(xb, Wv_ref[...]) + bv_ref[...]
        aw = _dot_nt(q, k) * (1.0 / _SCALE)
        aw = aw - jnp.max(aw, axis=1, keepdims=True)
        aw = jnp.exp(aw)
        aw = aw / jnp.sum(aw, axis=1, keepdims=True)
        out = _dot(aw, v)
        out = _gelu(_dot(out, W1_ref[...]) + b1_ref[...])
        out = _dot(out, W2_ref[...]) + b2_ref[...]
        xb = _layer_norm(xb + out, lnfs_ref[...], lnfb_ref[...])
        out_ref[b] = jnp.max(xb, axis=0, keepdims=True)


@functools.partial(jax.jit, static_argnames=())
def kernel(points, params):
    B, S, n, d_in = points.shape
    nc = B * S
    pts = points.reshape(nc, n, d_in)
    posT = jnp.swapaxes(pts[..., :ND], 1, 2)

    def stk(nm):
        return jnp.stack([params[f'l{i}_{nm}'] for i in range(L)])

    def stk_b(nm):
        return jnp.stack([params[f'l{i}_{nm}'] for i in range(L)]).reshape(L, 1, H)

    Wqkv = jnp.concatenate([stk('Wq'), stk('Wk'), stk('Wv')], axis=2)
    bqkv = jnp.concatenate(
        [stk_b('bq'), stk_b('bk'), stk_b('bv')], axis=2)
    Wo = stk('Wo')
    Wp = stk('Wp')
    WpdT = jnp.swapaxes(stk('Wpd'), 1, 2)
    wa = stk('Wa').reshape(L, 1, H) * (1.0 / _SCALE)
    bp, bo = stk_b('bp'), stk_b('bo')
    lns, lnb = stk_b('ln_scale'), stk_b('ln_bias')
    Win = params['W_in']
    bin_ = params['b_in'].reshape(1, H)

    def cloud_spec(shape):
        nd = len(shape)
        return pl.BlockSpec((1,) + shape[1:], lambda p: (p,) + (0,) * (nd - 1))

    def whole_spec(shape):
        nd = len(shape)
        return pl.BlockSpec(shape, lambda p: (0,) * nd)

    pooled = pl.pallas_call(
        _cloud_kernel,
        grid=(nc,),
        in_specs=[
            cloud_spec((nc, n, d_in)),
            cloud_spec((nc, ND, n)),
            whole_spec((d_in, H)), whole_spec((1, H)),
            whole_spec((L, H, 3 * H)), whole_spec((L, 1, 3 * H)),
            whole_spec((L, ND, H)), whole_spec((L, 1, H)),
            whole_spec((L, H, ND)),
            whole_spec((L, 1, H)),
            whole_spec((L, H, H)), whole_spec((L, 1, H)),
            whole_spec((L, 1, H)), whole_spec((L, 1, H)),
        ],
        out_specs=pl.BlockSpec((1, 1, H), lambda p: (p, 0, 0)),
        out_shape=jax.ShapeDtypeStruct((nc, 1, H), jnp.float32),
        scratch_shapes=[pltpu.VMEM((N, N), jnp.float32),
                        pltpu.VMEM((N, N), jnp.float32)],
        compiler_params=pltpu.CompilerParams(
            dimension_semantics=("parallel",)),
    )(pts, posT, Win, bin_, Wqkv, bqkv,
      Wp, bp, WpdT, wa, Wo, bo, lns, lnb)

    pooled = pooled.reshape(B, S, H)

    out = pl.pallas_call(
        _encoder_kernel,
        out_shape=jax.ShapeDtypeStruct((B, 1, H), jnp.float32),
    )(pooled,
      params['enc_Wq'], params['enc_Wk'], params['enc_Wv'],
      params['enc_W1'], params['enc_W2'],
      params['enc_bq'].reshape(1, H), params['enc_bk'].reshape(1, H),
      params['enc_bv'].reshape(1, H), params['enc_b1'].reshape(1, H),
      params['enc_b2'].reshape(1, H),
      params['lnf_scale'].reshape(1, H), params['lnf_bias'].reshape(1, H))
    return out.reshape(B, H)


# speedup vs baseline: 1.0307x; 1.0307x over previous
import functools
import math

import jax
import jax.numpy as jnp
from jax.experimental import pallas as pl
from jax.experimental.pallas import tpu as pltpu

N = 1024
H = 128
K = 16
L = 3
ND = 3
BIG = 1e30
_SCALE = math.sqrt(128.0)
_PREC = jax.lax.Precision.DEFAULT


def _dot(a, b, precision=_PREC):
    return jnp.dot(a, b, preferred_element_type=jnp.float32, precision=precision)


def _dot_nt(a, b, precision=_PREC):
    return jax.lax.dot_general(
        a, b, (((1,), (1,)), ((), ())),
        preferred_element_type=jnp.float32, precision=precision)


def _gelu(x):
    return jax.nn.gelu(x)


def _layer_norm(x, sc, bi):
    m = jnp.mean(x, axis=-1, keepdims=True)
    v = jnp.mean((x - m) ** 2, axis=-1, keepdims=True)
    return (x - m) / jnp.sqrt(v + 1e-6) * sc + bi


def _cloud_kernel(pts_ref, posT_ref, Win_ref, bin_ref,
                  Wqkv_ref, bqkv_ref,
                  Wp_ref, bp_ref, WpdT_ref, wa_ref,
                  Wo_ref, bo_ref, lns_ref, lnb_ref,
                  eWq_ref, eWk_ref, eWv_ref, eW1_ref, eW2_ref,
                  ebq_ref, ebk_ref, ebv_ref, eb1_ref, eb2_ref,
                  lnfs_ref, lnfb_ref,
                  out_ref, A_s, madd_s, pool_s):
    pid = pl.program_id(0)
    nc = pl.num_programs(0)
    pts = pts_ref[0]
    pos = pts[:, :ND]
    x = _dot(pts, Win_ref[...]) + bin_ref[...]

    px = pos[:, 0:1]
    py = pos[:, 1:2]
    pz = pos[:, 2:3]
    rx = posT_ref[0, 0:1, :]
    ry = posT_ref[0, 1:2, :]
    rz = posT_ref[0, 2:3, :]
    dx = px - rx
    dy = py - ry
    dz = pz - rz
    col = jax.lax.broadcasted_iota(jnp.int32, (N, N), 1)
    row = jax.lax.broadcasted_iota(jnp.int32, (N, N), 0)
    A_s[...] = jnp.where(col == row, BIG, dx * dx + dy * dy + dz * dz)

    def _select_one(_, carry):
        d = A_s[...]
        m = jnp.min(d, axis=1, keepdims=True)
        idx = jnp.min(jnp.where(d == m, col, N), axis=1, keepdims=True)
        A_s[...] = jnp.where(col == idx, BIG, d)
        return carry

    jax.lax.fori_loop(0, K - 1, _select_one, 0)
    madd_s[...] = jnp.where(A_s[...] == BIG, 0.0, -BIG)
    madd = madd_s[...]

    for i in range(L):
        qkv = _dot(x, Wqkv_ref[i]) + bqkv_ref[i]
        q = qkv[:, :H]
        k = qkv[:, H:2 * H]
        v = qkv[:, 2 * H:]
        pe = _dot(pos, Wp_ref[i]) + bp_ref[i]
        u = (q + pe) * wa_ref[i]
        kk = k + pe
        w = _dot(u, WpdT_ref[i])
        Araw = _dot_nt(u, kk)
        m = jnp.max(
            Araw + (w[:, 0:1] * rx + w[:, 1:2] * ry + w[:, 2:3] * rz) + madd,
            axis=1, keepdims=True)
        p_un = jnp.exp(
            Araw + (w[:, 0:1] * rx + w[:, 1:2] * ry + w[:, 2:3] * rz) + madd
            - m)
        ssum = jnp.sum(p_un, axis=1, keepdims=True)
        out = _dot(p_un, v) / ssum
        out = _dot(out, Wo_ref[i]) + bo_ref[i]
        x = x + _gelu(out)
        x = _layer_norm(x, lns_ref[i], lnb_ref[i])

    pool_s[pl.ds(pid, 1), :] = jnp.max(x, axis=0, keepdims=True)

    @pl.when(pid == nc - 1)
    def _run_encoder():
        nb = out_ref.shape[0]
        ns = pool_s.shape[0] // nb
        for b in range(nb):
            xb = pool_s[b * ns:(b + 1) * ns, :]
            q = _dot(xb, eWq_ref[...]) + ebq_ref[...]
            k = _dot(xb, eWk_ref[...]) + ebk_ref[...]
            v = _dot(xb, eWv_ref[...]) + ebv_ref[...]
            aw = _dot_nt(q, k) * (1.0 / _SCALE)
            aw = aw - jnp.max(aw, axis=1, keepdims=True)
            aw = jnp.exp(aw)
            aw = aw / jnp.sum(aw, axis=1, keepdims=True)
            out = _dot(aw, v)
            out = _gelu(_dot(out, eW1_ref[...]) + eb1_ref[...])
            out = _dot(out, eW2_ref[...]) + eb2_ref[...]
            xb = _layer_norm(xb + out, lnfs_ref[...], lnfb_ref[...])
            out_ref[b:b + 1, :] = jnp.max(xb, axis=0, keepdims=True)


@functools.partial(jax.jit, static_argnames=())
def kernel(points, params):
    B, S, n, d_in = points.shape
    nc = B * S
    pts = points.reshape(nc, n, d_in)
    posT = jnp.swapaxes(pts[..., :ND], 1, 2)

    def stk(nm):
        return jnp.stack([params[f'l{i}_{nm}'] for i in range(L)])

    def stk_b(nm):
        return jnp.stack([params[f'l{i}_{nm}'] for i in range(L)]).reshape(L, 1, H)

    Wqkv = jnp.concatenate([stk('Wq'), stk('Wk'), stk('Wv')], axis=2)
    bqkv = jnp.concatenate(
        [stk_b('bq'), stk_b('bk'), stk_b('bv')], axis=2)
    Wo = stk('Wo')
    Wp = stk('Wp')
    WpdT = jnp.swapaxes(stk('Wpd'), 1, 2)
    wa = stk('Wa').reshape(L, 1, H) * (1.0 / _SCALE)
    bp, bo = stk_b('bp'), stk_b('bo')
    lns, lnb = stk_b('ln_scale'), stk_b('ln_bias')
    Win = params['W_in']
    bin_ = params['b_in'].reshape(1, H)

    def cloud_spec(shape):
        nd = len(shape)
        return pl.BlockSpec((1,) + shape[1:], lambda p: (p,) + (0,) * (nd - 1))

    def whole_spec(shape):
        nd = len(shape)
        return pl.BlockSpec(shape, lambda p: (0,) * nd)

    out = pl.pallas_call(
        _cloud_kernel,
        grid=(nc,),
        in_specs=[
            cloud_spec((nc, n, d_in)),
            cloud_spec((nc, ND, n)),
            whole_spec((d_in, H)), whole_spec((1, H)),
            whole_spec((L, H, 3 * H)), whole_spec((L, 1, 3 * H)),
            whole_spec((L, ND, H)), whole_spec((L, 1, H)),
            whole_spec((L, H, ND)),
            whole_spec((L, 1, H)),
            whole_spec((L, H, H)), whole_spec((L, 1, H)),
            whole_spec((L, 1, H)), whole_spec((L, 1, H)),
            whole_spec((H, H)), whole_spec((H, H)), whole_spec((H, H)),
            whole_spec((H, H)), whole_spec((H, H)),
            whole_spec((1, H)), whole_spec((1, H)), whole_spec((1, H)),
            whole_spec((1, H)), whole_spec((1, H)),
            whole_spec((1, H)), whole_spec((1, H)),
        ],
        out_specs=pl.BlockSpec((B, H), lambda p: (0, 0)),
        out_shape=jax.ShapeDtypeStruct((B, H), jnp.float32),
        scratch_shapes=[pltpu.VMEM((N, N), jnp.float32),
                        pltpu.VMEM((N, N), jnp.float32),
                        pltpu.VMEM((nc, H), jnp.float32)],
        compiler_params=pltpu.CompilerParams(
            dimension_semantics=("arbitrary",)),
    )(pts, posT, Win, bin_, Wqkv, bqkv,
      Wp, bp, WpdT, wa, Wo, bo, lns, lnb,
      params['enc_Wq'], params['enc_Wk'], params['enc_Wv'],
      params['enc_W1'], params['enc_W2'],
      params['enc_bq'].reshape(1, H), params['enc_bk'].reshape(1, H),
      params['enc_bv'].reshape(1, H), params['enc_b1'].reshape(1, H),
      params['enc_b2'].reshape(1, H),
      params['lnf_scale'].reshape(1, H), params['lnf_bias'].reshape(1, H))
    return out
